# baseline (speedup 1.0000x reference)
import jax
import jax.numpy as jnp
from jax import lax
from jax.experimental import pallas as pl
from jax.experimental.pallas import tpu as pltpu
from jax.experimental.pallas import tpu_sc as plsc

N = 10000
F = 128
H2 = 16
C = 10
E = 320000

NC, NS = 2, 16
NW = NC * NS
L = 16

_MESH = plsc.VectorSubcoreMesh(core_axis_name="c", subcore_axis_name="s")
_SC_PARAMS = pltpu.CompilerParams(needs_layout_passes=False)

_MASK_LO = 0xFFFF
_MASK_HI = -65536


def _unpack2(v):
    vlo = plsc.bitcast(v << 16, jnp.float32)
    vhi = plsc.bitcast(v & _MASK_HI, jnp.float32)
    return vlo, vhi


EPT = E // NW
UD = 5


def _deg_body(src_hbm, dst_hbm, degp_hbm, epk_hbm, sbuf, dbuf, ebuf, hist,
              sem):
    wid = lax.axis_index("c") * NS + lax.axis_index("s")
    c1 = pltpu.async_copy(src_hbm.at[pl.ds(wid * EPT, EPT)], sbuf, sem)
    c2 = pltpu.async_copy(dst_hbm.at[pl.ds(wid * EPT, EPT)], dbuf, sem)
    zeros = jnp.zeros((L,), jnp.float32)

    @plsc.parallel_loop(0, N // L, unroll=5)
    def _(i):
        hist[pl.ds(i * L, L)] = zeros

    c1.wait()
    c2.wait()
    ones = jnp.ones((L,), jnp.float32)

    @plsc.parallel_loop(0, EPT // L, unroll=UD)
    def _(i):
        sl = pl.ds(i * L, L)
        s = sbuf[sl]
        d = dbuf[sl]
        ebuf[sl] = (d << 16) | s
        plsc.addupdate_scatter(hist, [d], ones)
    pltpu.sync_copy(hist, degp_hbm.at[pl.ds(wid * N, N)])
    pltpu.sync_copy(ebuf, epk_hbm.at[pl.ds(wid * EPT, EPT)])


_deg_kernel = pl.kernel(
    _deg_body,
    out_type=(
        jax.ShapeDtypeStruct((NW * N,), jnp.float32),
        jax.ShapeDtypeStruct((E,), jnp.int32),
    ),
    mesh=_MESH,
    compiler_params=_SC_PARAMS,
    scratch_types=[
        pltpu.VMEM((EPT,), jnp.int32),
        pltpu.VMEM((EPT,), jnp.int32),
        pltpu.VMEM((EPT,), jnp.int32),
        pltpu.VMEM((N,), jnp.float32),
        pltpu.SemaphoreType.DMA,
    ],
)

FPT = F // NW
CE1 = 16000
NCH1 = E // CE1
U1 = 8


def _feat1(w, j, k):
    return 2 * w + j + k * 64


def _spmm1_body(hT_hbm, hp_hbm, epk_hbm, out_hbm, table, acc, ebuf0, ebuf1,
                sem0, sem1):
    wid = lax.axis_index("c") * NS + lax.axis_index("s")
    pltpu.sync_copy(hp_hbm.at[pl.ds((2 * wid) * N, 2 * N)], table)
    for j in range(2):
        for k in range(2):
            pltpu.sync_copy(hT_hbm.at[pl.ds(_feat1(wid, j, k) * N, N)],
                            acc.at[pl.ds((2 * j + k) * N, N)])
    sems = (sem0, sem1)
    ebufs = (ebuf0, ebuf1)

    def start(ch):
        b = ch % 2
        return pltpu.async_copy(epk_hbm.at[pl.ds(ch * CE1, CE1)], ebufs[b],
                                sems[b])

    pend = start(0)
    for ch in range(NCH1):
        b = ch % 2
        pend.wait()
        if ch + 1 < NCH1:
            pend = start(ch + 1)
        eb = ebufs[b]

        @plsc.parallel_loop(0, CE1 // L, unroll=U1)
        def _(i, eb=eb):
            w_ = eb[pl.ds(i * L, L)]
            s = w_ & _MASK_LO
            d = lax.shift_right_logical(w_, 16)
            for j in range(2):
                v = plsc.load_gather(table, [s + j * N])
                vlo, vhi = _unpack2(v)
                plsc.addupdate_scatter(acc, [d + (2 * j) * N], vlo)
                plsc.addupdate_scatter(acc, [d + (2 * j + 1) * N], vhi)
    for j in range(2):
        for k in range(2):
            pltpu.sync_copy(acc.at[pl.ds((2 * j + k) * N, N)],
                            out_hbm.at[pl.ds(_feat1(wid, j, k) * N, N)])


_spmm1_kernel = pl.kernel(
    _spmm1_body,
    out_type=jax.ShapeDtypeStruct((F * N,), jnp.float32),
    mesh=_MESH,
    compiler_params=_SC_PARAMS,
    scratch_types=[
        pltpu.VMEM((2 * N,), jnp.int32),
        pltpu.VMEM((FPT * N,), jnp.float32),
        pltpu.VMEM((CE1,), jnp.int32),
        pltpu.VMEM((CE1,), jnp.int32),
        pltpu.SemaphoreType.DMA,
        pltpu.SemaphoreType.DMA,
    ],
)

G2 = H2 // FPT
NE2 = NW // G2
EH = E // NE2
CE2 = 8000
NCH2 = EH // CE2
U2 = 8


def _feat2(g, j, k):
    return 2 * g + j + k * 8


def _spmm2_body(hp_hbm, epk_hbm, out_hbm, table, acc, ebuf0, ebuf1,
                sem0, sem1, semt):
    wid = lax.axis_index("c") * NS + lax.axis_index("s")
    grp = wid % G2
    shard = wid // G2
    cpt = pltpu.async_copy(hp_hbm.at[pl.ds((2 * grp) * N, 2 * N)], table,
                           semt)
    ebase = shard * EH
    sems = (sem0, sem1)
    ebufs = (ebuf0, ebuf1)

    def start(ch):
        b = ch % 2
        return pltpu.async_copy(epk_hbm.at[pl.ds(ebase + ch * CE2, CE2)],
                                ebufs[b], sems[b])

    pend = start(0)
    zeros = jnp.zeros((L,), jnp.float32)

    @plsc.parallel_loop(0, (FPT * N) // L, unroll=5)
    def _(i):
        acc[pl.ds(i * L, L)] = zeros

    cpt.wait()
    for ch in range(NCH2):
        b = ch % 2
        pend.wait()
        if ch + 1 < NCH2:
            pend = start(ch + 1)
        eb = ebufs[b]

        @plsc.parallel_loop(0, CE2 // L, unroll=U2)
        def _(i, eb=eb):
            w_ = eb[pl.ds(i * L, L)]
            s = w_ & _MASK_LO
            d = lax.shift_right_logical(w_, 16)
            for j in range(2):
                v = plsc.load_gather(table, [s + j * N])
                vlo, vhi = _unpack2(v)
                plsc.addupdate_scatter(acc, [d + (2 * j) * N], vlo)
                plsc.addupdate_scatter(acc, [d + (2 * j + 1) * N], vhi)
    for j in range(2):
        for k in range(2):
            obase = (shard * H2 + _feat2(grp, j, k)) * N
            pltpu.sync_copy(acc.at[pl.ds((2 * j + k) * N, N)],
                            out_hbm.at[pl.ds(obase, N)])


_spmm2_kernel = pl.kernel(
    _spmm2_body,
    out_type=jax.ShapeDtypeStruct((NE2 * H2 * N,), jnp.float32),
    mesh=_MESH,
    compiler_params=_SC_PARAMS,
    scratch_types=[
        pltpu.VMEM((2 * N,), jnp.int32),
        pltpu.VMEM((FPT * N,), jnp.float32),
        pltpu.VMEM((CE2,), jnp.int32),
        pltpu.VMEM((CE2,), jnp.int32),
        pltpu.SemaphoreType.DMA,
        pltpu.SemaphoreType.DMA,
        pltpu.SemaphoreType.DMA,
    ],
)

def _dinv_of(degp_blk):
    deg = jnp.sum(degp_blk, axis=0) + 1.0
    return lax.rsqrt(deg)


def _pack_rows(h, half):
    lo = lax.bitcast_convert_type(h[0:half, :], jnp.uint32)
    hi = lax.bitcast_convert_type(h[half:2 * half, :], jnp.uint32)
    rnd = jnp.uint32(0x8000)
    lo = lax.shift_right_logical(lo + rnd, jnp.uint32(16))
    hi = (hi + rnd) & jnp.uint32(0xFFFF0000)
    return lax.bitcast_convert_type(hi | lo, jnp.int32)


def _mm1_body(x_ref, w1_ref, degp_ref, out_ref, outp_ref):
    dinv = _dinv_of(degp_ref[...])
    h = lax.dot_general(w1_ref[...], x_ref[...], (((0,), (1,)), ((), ())),
                        preferred_element_type=jnp.float32)
    h = h * dinv[None, :]
    out_ref[...] = h
    outp_ref[...] = _pack_rows(h, F // 2)


def _mm2_body(aggT_ref, degp_ref, w2T_ref, b1_ref, out_ref, outp_ref):
    dinv = _dinv_of(degp_ref[...])
    out1 = jnp.maximum(aggT_ref[...] * dinv[None, :] + b1_ref[...], 0.0)
    h = jnp.dot(w2T_ref[...], out1, preferred_element_type=jnp.float32)
    h = h * dinv[None, :]
    out_ref[...] = h
    outp_ref[...] = _pack_rows(h, H2 // 2)


def _mm3_body(p_ref, h2T_ref, degp_ref, b2_ref, fcWT_ref, fcb_ref,
              xe_ref, pr_ref):
    dinv = _dinv_of(degp_ref[...])
    agg = h2T_ref[...]
    for e in range(NE2):
        agg = agg + p_ref[e * H2:(e + 1) * H2, :]
    xe = agg * dinv[None, :] + b2_ref[...]
    xe_ref[...] = xe.T
    logits = jnp.dot(fcWT_ref[...], xe, preferred_element_type=jnp.float32)
    pr_ref[...] = jax.nn.sigmoid(logits + fcb_ref[...]).T


_mm1 = pl.pallas_call(
    _mm1_body,
    out_shape=[
        jax.ShapeDtypeStruct((F, N), jnp.float32),
        jax.ShapeDtypeStruct((F // 2, N), jnp.int32),
    ],
)

_mm2 = pl.pallas_call(
    _mm2_body,
    out_shape=[
        jax.ShapeDtypeStruct((H2, N), jnp.float32),
        jax.ShapeDtypeStruct((H2 // 2, N), jnp.int32),
    ],
)

_mm3 = pl.pallas_call(
    _mm3_body,
    out_shape=[
        jax.ShapeDtypeStruct((N, H2), jnp.float32),
        jax.ShapeDtypeStruct((N, C), jnp.float32),
    ],
)


@jax.jit
def kernel(x, edge_index, W1, b1, W2, b2, fc_W, fc_b):
    src = edge_index[0].astype(jnp.int32)
    dst = edge_index[1].astype(jnp.int32)

    degp_flat, epk = _deg_kernel(src, dst)
    degp = degp_flat.reshape(NW, N)

    h1T, h1p = _mm1(x, W1, degp)
    agg1T = _spmm1_kernel(h1T.reshape(-1), h1p.reshape(-1), epk).reshape(F, N)
    h2T, h2p = _mm2(agg1T, degp, W2.T, b1.reshape(F, 1))
    p2 = _spmm2_kernel(h2p.reshape(-1), epk).reshape(NE2 * H2, N)
    x_emb, probs = _mm3(p2, h2T, degp, b2.reshape(H2, 1),
                        fc_W.T, fc_b.reshape(C, 1))
    return (x_emb, probs)


# speedup vs baseline: 1.0491x; 1.0491x over previous
import jax
import jax.numpy as jnp
from jax import lax
from jax.experimental import pallas as pl
from jax.experimental.pallas import tpu as pltpu
from jax.experimental.pallas import tpu_sc as plsc

N = 10000
F = 128
H2 = 16
C = 10
E = 320000

NC, NS = 2, 16
NW = NC * NS
L = 16

_MESH = plsc.VectorSubcoreMesh(core_axis_name="c", subcore_axis_name="s")
_SC_PARAMS = pltpu.CompilerParams(needs_layout_passes=False)

_MASK_LO = 0xFFFF
_MASK_HI = -65536


def _unpack2(v):
    vlo = plsc.bitcast(v << 16, jnp.float32)
    vhi = plsc.bitcast(v & _MASK_HI, jnp.float32)
    return vlo, vhi


EPT = E // NW
UD = 5


def _deg_body(src_hbm, dst_hbm, degp_hbm, epk_hbm, sbuf, dbuf, ebuf, hist,
              sem):
    wid = lax.axis_index("c") * NS + lax.axis_index("s")
    c1 = pltpu.async_copy(src_hbm.at[pl.ds(wid * EPT, EPT)], sbuf, sem)
    c2 = pltpu.async_copy(dst_hbm.at[pl.ds(wid * EPT, EPT)], dbuf, sem)
    zeros = jnp.zeros((L,), jnp.float32)

    @plsc.parallel_loop(0, N // L, unroll=5)
    def _(i):
        hist[pl.ds(i * L, L)] = zeros

    c1.wait()
    c2.wait()
    ones = jnp.ones((L,), jnp.float32)

    @plsc.parallel_loop(0, EPT // L, unroll=UD)
    def _(i):
        sl = pl.ds(i * L, L)
        s = sbuf[sl]
        d = dbuf[sl]
        ebuf[sl] = (d << 16) | s
        plsc.addupdate_scatter(hist, [d], ones)
    pltpu.sync_copy(hist, degp_hbm.at[pl.ds(wid * N, N)])
    pltpu.sync_copy(ebuf, epk_hbm.at[pl.ds(wid * EPT, EPT)])


_deg_kernel = pl.kernel(
    _deg_body,
    out_type=(
        jax.ShapeDtypeStruct((NW * N,), jnp.float32),
        jax.ShapeDtypeStruct((E,), jnp.int32),
    ),
    mesh=_MESH,
    compiler_params=_SC_PARAMS,
    scratch_types=[
        pltpu.VMEM((EPT,), jnp.int32),
        pltpu.VMEM((EPT,), jnp.int32),
        pltpu.VMEM((EPT,), jnp.int32),
        pltpu.VMEM((N,), jnp.float32),
        pltpu.SemaphoreType.DMA,
    ],
)

FPT = F // NW
CE1 = 16000
NCH1 = E // CE1
U1 = 4


def _feat1(w, j, k):
    return 2 * w + j + k * 64


def _spmm1_body(hT_hbm, hp_hbm, epk_hbm, out_hbm, table, acc, ebuf0, ebuf1,
                sem0, sem1):
    wid = lax.axis_index("c") * NS + lax.axis_index("s")
    pltpu.sync_copy(hp_hbm.at[pl.ds((2 * wid) * N, 2 * N)], table)
    for j in range(2):
        for k in range(2):
            pltpu.sync_copy(hT_hbm.at[pl.ds(_feat1(wid, j, k) * N, N)],
                            acc.at[pl.ds((2 * j + k) * N, N)])
    sems = (sem0, sem1)
    ebufs = (ebuf0, ebuf1)

    def start(ch):
        b = ch % 2
        return pltpu.async_copy(epk_hbm.at[pl.ds(ch * CE1, CE1)], ebufs[b],
                                sems[b])

    pend = start(0)
    for ch in range(NCH1):
        b = ch % 2
        pend.wait()
        if ch + 1 < NCH1:
            pend = start(ch + 1)
        eb = ebufs[b]

        @plsc.parallel_loop(0, CE1 // L, unroll=U1)
        def _(i, eb=eb):
            w_ = eb[pl.ds(i * L, L)]
            s = w_ & _MASK_LO
            d = lax.shift_right_logical(w_, 16)
            for j in range(2):
                v = plsc.load_gather(table, [s + j * N])
                vlo, vhi = _unpack2(v)
                plsc.addupdate_scatter(acc, [d + (2 * j) * N], vlo)
                plsc.addupdate_scatter(acc, [d + (2 * j + 1) * N], vhi)
    for j in range(2):
        for k in range(2):
            pltpu.sync_copy(acc.at[pl.ds((2 * j + k) * N, N)],
                            out_hbm.at[pl.ds(_feat1(wid, j, k) * N, N)])


_spmm1_kernel = pl.kernel(
    _spmm1_body,
    out_type=jax.ShapeDtypeStruct((F * N,), jnp.float32),
    mesh=_MESH,
    compiler_params=_SC_PARAMS,
    scratch_types=[
        pltpu.VMEM((2 * N,), jnp.int32),
        pltpu.VMEM((FPT * N,), jnp.float32),
        pltpu.VMEM((CE1,), jnp.int32),
        pltpu.VMEM((CE1,), jnp.int32),
        pltpu.SemaphoreType.DMA,
        pltpu.SemaphoreType.DMA,
    ],
)

G2 = H2 // FPT
NE2 = NW // G2
EH = E // NE2
CE2 = 8000
NCH2 = EH // CE2
U2 = 8


def _feat2(g, j, k):
    return 2 * g + j + k * 8


def _spmm2_body(hp_hbm, epk_hbm, out_hbm, table, acc, ebuf0, ebuf1,
                sem0, sem1, semt):
    wid = lax.axis_index("c") * NS + lax.axis_index("s")
    grp = wid % G2
    shard = wid // G2
    cpt = pltpu.async_copy(hp_hbm.at[pl.ds((2 * grp) * N, 2 * N)], table,
                           semt)
    ebase = shard * EH
    sems = (sem0, sem1)
    ebufs = (ebuf0, ebuf1)

    def start(ch):
        b = ch % 2
        return pltpu.async_copy(epk_hbm.at[pl.ds(ebase + ch * CE2, CE2)],
                                ebufs[b], sems[b])

    pend = start(0)
    zeros = jnp.zeros((L,), jnp.float32)

    @plsc.parallel_loop(0, (FPT * N) // L, unroll=5)
    def _(i):
        acc[pl.ds(i * L, L)] = zeros

    cpt.wait()
    for ch in range(NCH2):
        b = ch % 2
        pend.wait()
        if ch + 1 < NCH2:
            pend = start(ch + 1)
        eb = ebufs[b]

        @plsc.parallel_loop(0, CE2 // L, unroll=U2)
        def _(i, eb=eb):
            w_ = eb[pl.ds(i * L, L)]
            s = w_ & _MASK_LO
            d = lax.shift_right_logical(w_, 16)
            for j in range(2):
                v = plsc.load_gather(table, [s + j * N])
                vlo, vhi = _unpack2(v)
                plsc.addupdate_scatter(acc, [d + (2 * j) * N], vlo)
                plsc.addupdate_scatter(acc, [d + (2 * j + 1) * N], vhi)
    for j in range(2):
        for k in range(2):
            obase = (shard * H2 + _feat2(grp, j, k)) * N
            pltpu.sync_copy(acc.at[pl.ds((2 * j + k) * N, N)],
                            out_hbm.at[pl.ds(obase, N)])


_spmm2_kernel = pl.kernel(
    _spmm2_body,
    out_type=jax.ShapeDtypeStruct((NE2 * H2 * N,), jnp.float32),
    mesh=_MESH,
    compiler_params=_SC_PARAMS,
    scratch_types=[
        pltpu.VMEM((2 * N,), jnp.int32),
        pltpu.VMEM((FPT * N,), jnp.float32),
        pltpu.VMEM((CE2,), jnp.int32),
        pltpu.VMEM((CE2,), jnp.int32),
        pltpu.SemaphoreType.DMA,
        pltpu.SemaphoreType.DMA,
        pltpu.SemaphoreType.DMA,
    ],
)

def _dinv_of(degp_blk):
    deg = jnp.sum(degp_blk, axis=0) + 1.0
    return lax.rsqrt(deg)


def _pack_rows(h, half):
    lo = lax.bitcast_convert_type(h[0:half, :], jnp.uint32)
    hi = lax.bitcast_convert_type(h[half:2 * half, :], jnp.uint32)
    rnd = jnp.uint32(0x8000)
    lo = lax.shift_right_logical(lo + rnd, jnp.uint32(16))
    hi = (hi + rnd) & jnp.uint32(0xFFFF0000)
    return lax.bitcast_convert_type(hi | lo, jnp.int32)


def _mm1_body(xT_ref, w1T_ref, degp_ref, out_ref, outp_ref):
    dinv = _dinv_of(degp_ref[...])
    h = jnp.dot(w1T_ref[...], xT_ref[...], preferred_element_type=jnp.float32)
    h = h * dinv[None, :]
    out_ref[...] = h
    outp_ref[...] = _pack_rows(h, F // 2)


def _mm2_body(aggT_ref, degp_ref, w2T_ref, b1_ref, out_ref, outp_ref):
    dinv = _dinv_of(degp_ref[...])
    out1 = jnp.maximum(aggT_ref[...] * dinv[None, :] + b1_ref[...], 0.0)
    h = jnp.dot(w2T_ref[...], out1, preferred_element_type=jnp.float32)
    h = h * dinv[None, :]
    out_ref[...] = h
    outp_ref[...] = _pack_rows(h, H2 // 2)


def _mm3_body(p_ref, h2T_ref, degp_ref, b2_ref, fcWT_ref, fcb_ref,
              xe_ref, pr_ref):
    dinv = _dinv_of(degp_ref[...])
    agg = h2T_ref[...]
    for e in range(NE2):
        agg = agg + p_ref[e * H2:(e + 1) * H2, :]
    xe = agg * dinv[None, :] + b2_ref[...]
    xe_ref[...] = xe
    logits = jnp.dot(fcWT_ref[...], xe, preferred_element_type=jnp.float32)
    pr_ref[...] = jax.nn.sigmoid(logits + fcb_ref[...])


_mm1 = pl.pallas_call(
    _mm1_body,
    out_shape=[
        jax.ShapeDtypeStruct((F, N), jnp.float32),
        jax.ShapeDtypeStruct((F // 2, N), jnp.int32),
    ],
)

_mm2 = pl.pallas_call(
    _mm2_body,
    out_shape=[
        jax.ShapeDtypeStruct((H2, N), jnp.float32),
        jax.ShapeDtypeStruct((H2 // 2, N), jnp.int32),
    ],
)

_mm3 = pl.pallas_call(
    _mm3_body,
    out_shape=[
        jax.ShapeDtypeStruct((H2, N), jnp.float32),
        jax.ShapeDtypeStruct((C, N), jnp.float32),
    ],
)


@jax.jit
def kernel(x, edge_index, W1, b1, W2, b2, fc_W, fc_b):
    src = edge_index[0].astype(jnp.int32)
    dst = edge_index[1].astype(jnp.int32)

    degp_flat, epk = _deg_kernel(src, dst)
    degp = degp_flat.reshape(NW, N)

    h1T, h1p = _mm1(x.T, W1.T, degp)
    agg1T = _spmm1_kernel(h1T.reshape(-1), h1p.reshape(-1), epk).reshape(F, N)
    h2T, h2p = _mm2(agg1T, degp, W2.T, b1.reshape(F, 1))
    p2 = _spmm2_kernel(h2p.reshape(-1), epk).reshape(NE2 * H2, N)
    x_embT, probsT = _mm3(p2, h2T, degp, b2.reshape(H2, 1),
                          fc_W.T, fc_b.reshape(C, 1))
    return (x_embT.T, probsT.T)
